# baseline (speedup 1.0000x reference)
import functools

import jax
import jax.numpy as jnp
from jax import lax
from jax.experimental import pallas as pl
from jax.experimental.pallas import tpu as pltpu
from jax.experimental.pallas import tpu_sc as plsc

EMBED = 64

_NC = 2
_NS = 16
_NW = _NC * _NS

_CHUNK = 128
_NB = 10
_LOOKAHEAD = 5


@functools.lru_cache(maxsize=None)
def _make_gather(B, V):
    assert B % (_NW * _CHUNK) == 0
    n_chunks = B // (_NW * _CHUNK)
    b_per_w = n_chunks * _CHUNK
    assert n_chunks >= _NB

    mesh = plsc.VectorSubcoreMesh(core_axis_name="c", subcore_axis_name="s")

    @functools.partial(
        pl.kernel,
        out_type=jax.ShapeDtypeStruct((B, EMBED), jnp.float32),
        mesh=mesh,
        compiler_params=pltpu.CompilerParams(use_tc_tiling_on_sc=False),
        scratch_types=[
            pltpu.VMEM((n_chunks, _CHUNK), jnp.int32),
            [pltpu.VMEM((_CHUNK, EMBED), jnp.float32)
             for _ in range(_NB)],
            [pltpu.SemaphoreType.DMA for _ in range(_NB)],
            [pltpu.SemaphoreType.DMA for _ in range(_NB)],
        ],
    )
    def gather_kernel(idx_hbm, table_hbm, out_hbm, ids_v, bufs,
                      gsems, wsems):
        wid = lax.axis_index("s") * _NC + lax.axis_index("c")
        base_row = wid * b_per_w
        pltpu.sync_copy(idx_hbm.at[wid], ids_v)

        def start_gather(j, slot):
            pltpu.async_copy(table_hbm.at[ids_v.at[j]], bufs[slot],
                             gsems[slot])

        def wait_gather(j, slot):
            pltpu.make_async_copy(table_hbm.at[ids_v.at[j]], bufs[slot],
                                  gsems[slot]).wait()

        def start_write(j, slot):
            pltpu.async_copy(
                bufs[slot],
                out_hbm.at[pl.ds(base_row + j * _CHUNK, _CHUNK)],
                wsems[slot])

        def wait_write(j, slot):
            pltpu.make_async_copy(
                bufs[slot],
                out_hbm.at[pl.ds(base_row + j * _CHUNK, _CHUNK)],
                wsems[slot]).wait()

        for j in range(_LOOKAHEAD):
            start_gather(j, j % _NB)

        for j in range(n_chunks):
            wait_gather(j, j % _NB)
            start_write(j, j % _NB)
            jn = j + _LOOKAHEAD
            if jn < n_chunks:
                jw = jn - _NB
                if jw >= 0:
                    wait_write(jw, jn % _NB)
                start_gather(jn, jn % _NB)

        for j in range(max(0, n_chunks - _NB), n_chunks):
            wait_write(j, j % _NB)

    return gather_kernel


def kernel(x, token_table, codebook):
    batch, length = x.shape
    B = batch * length
    V, D = token_table.shape
    idx = x.astype(jnp.int32).reshape(_NW, B // (_NW * _CHUNK), _CHUNK)
    out = _make_gather(B, V)(idx, token_table)
    return out.reshape(batch, length, D)
